# baseline (speedup 1.0000x reference)
import functools

import jax
import jax.numpy as jnp
from jax import lax
from jax.experimental import pallas as pl
from jax.experimental.pallas import tpu as pltpu
from jax.experimental.pallas import tpu_sc as plsc

N = 10000
E = 320000
D = 128

NC = 2
NS = 16
L = 16
NW = NC * NS

K = 80
CHUNKS = E // NW // K
EPT = K * CHUNKS

NPAD = 10112
ROWS_PER_TILE = NPAD // NS

_LO_COLS = [c for c in range(D) if (c // 16) % 2 == 0]
_HI_COLS = [c for c in range(D) if (c // 16) % 2 == 1]
DW = D // 2
NPAD1 = 10240
DEG_PER_TILE = NPAD1 // NS

_sc_mesh = plsc.VectorSubcoreMesh(
    core_axis_name="c", subcore_axis_name="s", num_cores=NC, num_subcores=NS
)


@functools.partial(
    pl.kernel,
    out_type=jax.ShapeDtypeStruct((NC, NPAD1), jnp.float32),
    mesh=_sc_mesh,
    scratch_types=[
        pltpu.VMEM((CHUNKS, K), jnp.int32),
        pltpu.VMEM((K,), jnp.float32),
        pltpu.VMEM((DEG_PER_TILE,), jnp.float32),
        pltpu.VMEM_SHARED((NPAD1,), jnp.float32),
    ],
)
def _sc_deg(dst_hbm, out_hbm, dst_v, ones_v, zeros_v, acc_sh):
    c = lax.axis_index("c")
    s = lax.axis_index("s")
    wid = s * NC + c
    pltpu.sync_copy(dst_hbm.at[wid], dst_v)

    def fill_ones(i, carry):
        ones_v[pl.ds(i * L, L)] = jnp.ones((L,), jnp.float32)
        return carry

    def fill_zeros(i, carry):
        zeros_v[pl.ds(i * L, L)] = jnp.zeros((L,), jnp.float32)
        return carry

    lax.fori_loop(0, K // L, fill_ones, 0)
    lax.fori_loop(0, DEG_PER_TILE // L, fill_zeros, 0)

    base = s * DEG_PER_TILE
    pltpu.sync_copy(zeros_v, acc_sh.at[pl.ds(base, DEG_PER_TILE)])
    plsc.subcore_barrier()

    def body(j, carry):
        pltpu.sync_copy(ones_v, acc_sh.at[dst_v.at[j]], add=True)
        return carry

    lax.fori_loop(0, CHUNKS, body, 0)
    plsc.subcore_barrier()
    pltpu.sync_copy(acc_sh.at[pl.ds(base, DEG_PER_TILE)],
                    out_hbm.at[c, pl.ds(base, DEG_PER_TILE)])


@functools.partial(
    pl.kernel,
    out_type=jax.ShapeDtypeStruct((NC, NPAD, D), jnp.float32),
    mesh=_sc_mesh,
    scratch_types=[
        pltpu.VMEM((EPT,), jnp.int32),
        pltpu.VMEM((CHUNKS, K), jnp.int32),
        pltpu.VMEM((K, DW), jnp.int32),
        pltpu.VMEM((K, DW), jnp.int32),
        pltpu.VMEM((K, D), jnp.float32),
        pltpu.VMEM_SHARED((NPAD, D), jnp.float32),
        pltpu.SemaphoreType.DMA,
        pltpu.SemaphoreType.DMA,
    ],
    compiler_params=pltpu.CompilerParams(use_tc_tiling_on_sc=False),
)
def _sc_agg(h_hbm, src_hbm, dst_hbm, out_hbm, src_v, dst_v, bf0_v, bf1_v,
            rows_v, acc_sh, sem0, sem1):
    c = lax.axis_index("c")
    s = lax.axis_index("s")
    wid = s * NC + c
    pltpu.sync_copy(src_hbm.at[wid], src_v)
    pltpu.sync_copy(dst_hbm.at[wid], dst_v)

    def zrow(r, carry):
        for q in range(D // L):
            rows_v[r, pl.ds(q * L, L)] = jnp.zeros((L,), jnp.float32)
        return carry

    lax.fori_loop(0, K, zrow, 0)

    base = s * ROWS_PER_TILE
    nfull = ROWS_PER_TILE // K
    for t in range(nfull):
        pltpu.sync_copy(rows_v, acc_sh.at[pl.ds(base + t * K, K)])
    rem = ROWS_PER_TILE - nfull * K
    if rem:
        pltpu.sync_copy(rows_v.at[pl.ds(0, rem)],
                        acc_sh.at[pl.ds(base + nfull * K, rem)])
    plsc.subcore_barrier()

    def g_start(j, buf, sem):
        pltpu.async_copy(h_hbm.at[src_v.at[pl.ds(j * K, K)]], buf, sem)

    def g_wait(j, buf, sem):
        pltpu.make_async_copy(h_hbm.at[src_v.at[pl.ds(j * K, K)]], buf, sem).wait()

    def convert(buf):
        def crow(r, carry):
            for q in range(D // 32):
                w = buf[r, pl.ds(q * L, L)]
                lo = lax.bitcast_convert_type(w << 16, jnp.float32)
                hi = lax.bitcast_convert_type(w & jnp.int32(-65536), jnp.float32)
                rows_v[r, pl.ds(q * 32, L)] = lo
                rows_v[r, pl.ds(q * 32 + L, L)] = hi
            return carry

        lax.fori_loop(0, K, crow, 0)

    def scat(j):
        pltpu.sync_copy(rows_v, acc_sh.at[dst_v.at[j]], add=True)

    g_start(0, bf0_v, sem0)

    def body(p, carry):
        j0 = 2 * p
        j1 = j0 + 1
        g_start(j1, bf1_v, sem1)
        g_wait(j0, bf0_v, sem0)
        convert(bf0_v)
        g_start(j0 + 2, bf0_v, sem0)
        scat(j0)
        g_wait(j1, bf1_v, sem1)
        convert(bf1_v)
        scat(j1)
        return carry

    lax.fori_loop(0, CHUNKS // 2, body, 0)
    g_wait(CHUNKS - 1, bf0_v, sem0)
    convert(bf0_v)
    scat(CHUNKS - 1)
    plsc.subcore_barrier()
    pltpu.sync_copy(acc_sh.at[pl.ds(base, ROWS_PER_TILE)],
                    out_hbm.at[c, pl.ds(base, ROWS_PER_TILE)])


BR = 2000
GRID = N // BR


def _dinv_of(deg_blk):
    return 1.0 / jnp.sqrt(deg_blk[:, 0:1] + deg_blk[:, 1:2] + 1.0)


def _pack_rows(lo, hi):
    lo_u = lax.bitcast_convert_type(lo, jnp.uint32) + jnp.uint32(0x8000)
    hi_u = lax.bitcast_convert_type(hi, jnp.uint32) + jnp.uint32(0x8000)
    word = (lo_u >> 16) | (hi_u & jnp.uint32(0xFFFF0000))
    return lax.bitcast_convert_type(word, jnp.int32)


def _mm1_body(x_ref, w_ref, wlo_ref, whi_ref, deg_ref, o_ref, oq_ref):
    dinv = _dinv_of(deg_ref[...])
    x = x_ref[...]
    o_ref[...] = jnp.dot(x, w_ref[...], preferred_element_type=jnp.float32) * dinv
    lo = jnp.dot(x, wlo_ref[...], preferred_element_type=jnp.float32) * dinv
    hi = jnp.dot(x, whi_ref[...], preferred_element_type=jnp.float32) * dinv
    oq_ref[...] = _pack_rows(lo, hi)


_mm1 = pl.pallas_call(
    _mm1_body,
    grid=(GRID,),
    in_specs=[
        pl.BlockSpec((BR, D), lambda i: (i, 0)),
        pl.BlockSpec((D, D), lambda i: (0, 0)),
        pl.BlockSpec((D, DW), lambda i: (0, 0)),
        pl.BlockSpec((D, DW), lambda i: (0, 0)),
        pl.BlockSpec((BR, 2), lambda i: (i, 0)),
    ],
    out_specs=[
        pl.BlockSpec((BR, D), lambda i: (i, 0)),
        pl.BlockSpec((BR, DW), lambda i: (i, 0)),
    ],
    out_shape=[
        jax.ShapeDtypeStruct((N, D), jnp.float32),
        jax.ShapeDtypeStruct((N, DW), jnp.int32),
    ],
)


def _mm2_body(a0_ref, a1_ref, h1_ref, deg_ref, b1_ref, w2_ref, w2lo_ref,
              w2hi_ref, o_ref, oq_ref):
    dinv = _dinv_of(deg_ref[...])
    t = (a0_ref[0] + a1_ref[0] + h1_ref[...]) * dinv + b1_ref[...]
    t = jnp.maximum(t, 0.0)
    o_ref[...] = jnp.dot(t, w2_ref[...], preferred_element_type=jnp.float32) * dinv
    lo = jnp.dot(t, w2lo_ref[...], preferred_element_type=jnp.float32) * dinv
    hi = jnp.dot(t, w2hi_ref[...], preferred_element_type=jnp.float32) * dinv
    oq_ref[...] = _pack_rows(lo, hi)


_mm2 = pl.pallas_call(
    _mm2_body,
    grid=(GRID,),
    in_specs=[
        pl.BlockSpec((1, BR, D), lambda i: (0, i, 0)),
        pl.BlockSpec((1, BR, D), lambda i: (1, i, 0)),
        pl.BlockSpec((BR, D), lambda i: (i, 0)),
        pl.BlockSpec((BR, 2), lambda i: (i, 0)),
        pl.BlockSpec((1, D), lambda i: (0, 0)),
        pl.BlockSpec((D, D), lambda i: (0, 0)),
        pl.BlockSpec((D, DW), lambda i: (0, 0)),
        pl.BlockSpec((D, DW), lambda i: (0, 0)),
    ],
    out_specs=[
        pl.BlockSpec((BR, D), lambda i: (i, 0)),
        pl.BlockSpec((BR, DW), lambda i: (i, 0)),
    ],
    out_shape=[
        jax.ShapeDtypeStruct((N, D), jnp.float32),
        jax.ShapeDtypeStruct((N, DW), jnp.int32),
    ],
)


def _fin_body(a0_ref, a1_ref, h2_ref, deg_ref, b2_ref, o_ref):
    dinv = _dinv_of(deg_ref[...])
    o_ref[...] = (a0_ref[0] + a1_ref[0] + h2_ref[...]) * dinv + b2_ref[...]


_fin = pl.pallas_call(
    _fin_body,
    grid=(GRID,),
    in_specs=[
        pl.BlockSpec((1, BR, D), lambda i: (0, i, 0)),
        pl.BlockSpec((1, BR, D), lambda i: (1, i, 0)),
        pl.BlockSpec((BR, D), lambda i: (i, 0)),
        pl.BlockSpec((BR, 2), lambda i: (i, 0)),
        pl.BlockSpec((1, D), lambda i: (0, 0)),
    ],
    out_specs=pl.BlockSpec((BR, D), lambda i: (i, 0)),
    out_shape=jax.ShapeDtypeStruct((N, D), jnp.float32),
)


def kernel(node_emb, edge_index, W1, b1, W2, b2):
    src_p = edge_index[0].reshape(NW, EPT)
    dst_p = edge_index[1].reshape(NW, CHUNKS, K)
    lo_cols = jnp.asarray(_LO_COLS, dtype=jnp.int32)
    hi_cols = jnp.asarray(_HI_COLS, dtype=jnp.int32)
    W1lo, W1hi = W1[:, lo_cols], W1[:, hi_cols]
    W2lo, W2hi = W2[:, lo_cols], W2[:, hi_cols]

    deg_parts = _sc_deg(dst_p)
    deg_t = deg_parts.T

    b1r = b1.reshape(1, D)
    b2r = b2.reshape(1, D)

    h1p, h1q = _mm1(node_emb, W1, W1lo, W1hi, deg_t)
    agg1 = _sc_agg(h1q, src_p, dst_p)
    h2p, h2q = _mm2(agg1, agg1, h1p, deg_t, b1r, W2, W2lo, W2hi)
    agg2 = _sc_agg(h2q, src_p, dst_p)
    out = _fin(agg2, agg2, h2p, deg_t, b2r)
    return out


# speedup vs baseline: 1.8770x; 1.8770x over previous
import functools

import jax
import jax.numpy as jnp
from jax import lax
from jax.experimental import pallas as pl
from jax.experimental.pallas import tpu as pltpu
from jax.experimental.pallas import tpu_sc as plsc

N = 10000
E = 320000
D = 128

NC = 2
NS = 16
L = 16
NW = NC * NS

K = 80
CHUNKS = E // NW // K
EPT = K * CHUNKS

NPAD = 10112
ROWS_PER_TILE = NPAD // NS
NPAD1 = 10240
DEG_PER_TILE = NPAD1 // NS

_sc_mesh = plsc.VectorSubcoreMesh(
    core_axis_name="c", subcore_axis_name="s", num_cores=NC, num_subcores=NS
)


@functools.partial(
    pl.kernel,
    out_type=jax.ShapeDtypeStruct((NC, NPAD1), jnp.float32),
    mesh=_sc_mesh,
    scratch_types=[
        pltpu.VMEM((CHUNKS, K), jnp.int32),
        pltpu.VMEM((K,), jnp.float32),
        pltpu.VMEM((DEG_PER_TILE,), jnp.float32),
        pltpu.VMEM_SHARED((NPAD1,), jnp.float32),
    ],
)
def _sc_deg(dst_hbm, out_hbm, dst_v, ones_v, zeros_v, acc_sh):
    c = lax.axis_index("c")
    s = lax.axis_index("s")
    wid = s * NC + c
    pltpu.sync_copy(dst_hbm.at[wid], dst_v)

    def fill_ones(i, carry):
        ones_v[pl.ds(i * L, L)] = jnp.ones((L,), jnp.float32)
        return carry

    def fill_zeros(i, carry):
        zeros_v[pl.ds(i * L, L)] = jnp.zeros((L,), jnp.float32)
        return carry

    lax.fori_loop(0, K // L, fill_ones, 0)
    lax.fori_loop(0, DEG_PER_TILE // L, fill_zeros, 0)

    base = s * DEG_PER_TILE
    pltpu.sync_copy(zeros_v, acc_sh.at[pl.ds(base, DEG_PER_TILE)])
    plsc.subcore_barrier()

    def body(j, carry):
        pltpu.sync_copy(ones_v, acc_sh.at[dst_v.at[j]], add=True)
        return carry

    lax.fori_loop(0, CHUNKS, body, 0)
    plsc.subcore_barrier()
    pltpu.sync_copy(acc_sh.at[pl.ds(base, DEG_PER_TILE)],
                    out_hbm.at[c, pl.ds(base, DEG_PER_TILE)])


@functools.partial(
    pl.kernel,
    out_type=jax.ShapeDtypeStruct((NC, NPAD, D), jnp.float32),
    mesh=_sc_mesh,
    scratch_types=[
        pltpu.VMEM((EPT,), jnp.int32),
        pltpu.VMEM((CHUNKS, K), jnp.int32),
        pltpu.VMEM((K, D), jnp.float32),
        pltpu.VMEM((K, D), jnp.float32),
        pltpu.VMEM_SHARED((NPAD, D), jnp.float32),
        pltpu.SemaphoreType.DMA,
        pltpu.SemaphoreType.DMA,
    ],
)
def _sc_agg(h_hbm, src_hbm, dst_hbm, out_hbm, src_v, dst_v, rows0_v, rows1_v,
            acc_sh, sem0, sem1):
    c = lax.axis_index("c")
    s = lax.axis_index("s")
    wid = s * NC + c
    pltpu.sync_copy(src_hbm.at[wid], src_v)
    pltpu.sync_copy(dst_hbm.at[wid], dst_v)

    def g_start(j, buf, sem):
        pltpu.async_copy(h_hbm.at[src_v.at[pl.ds(j * K, K)]], buf, sem)

    def g_wait(j, buf, sem):
        pltpu.make_async_copy(h_hbm.at[src_v.at[pl.ds(j * K, K)]], buf, sem).wait()

    def scat(j, buf):
        pltpu.sync_copy(buf, acc_sh.at[dst_v.at[j]], add=True)

    g_start(0, rows0_v, sem0)

    def zrow(r, carry):
        for q in range(D // L):
            rows1_v[r, pl.ds(q * L, L)] = jnp.zeros((L,), jnp.float32)
        return carry

    lax.fori_loop(0, K, zrow, 0)

    base = s * ROWS_PER_TILE
    nfull = ROWS_PER_TILE // K
    for t in range(nfull):
        pltpu.sync_copy(rows1_v, acc_sh.at[pl.ds(base + t * K, K)])
    rem = ROWS_PER_TILE - nfull * K
    if rem:
        pltpu.sync_copy(rows1_v.at[pl.ds(0, rem)],
                        acc_sh.at[pl.ds(base + nfull * K, rem)])
    plsc.subcore_barrier()

    def body(p, carry):
        j0 = 2 * p
        j1 = j0 + 1
        g_start(j1, rows1_v, sem1)
        g_wait(j0, rows0_v, sem0)
        scat(j0, rows0_v)
        g_start(j0 + 2, rows0_v, sem0)
        g_wait(j1, rows1_v, sem1)
        scat(j1, rows1_v)
        return carry

    lax.fori_loop(0, CHUNKS // 2, body, 0)
    g_wait(CHUNKS - 1, rows0_v, sem0)
    scat(CHUNKS - 1, rows0_v)
    plsc.subcore_barrier()
    pltpu.sync_copy(acc_sh.at[pl.ds(base, ROWS_PER_TILE)],
                    out_hbm.at[c, pl.ds(base, ROWS_PER_TILE)])


BR = 1000
GRID = N // BR


def _dinv_of(deg_blk):
    return 1.0 / jnp.sqrt(deg_blk[:, 0:1] + deg_blk[:, 1:2] + 1.0)


def _mm1_body(x_ref, w_ref, deg_ref, o_ref):
    dinv = _dinv_of(deg_ref[...])
    h = jnp.dot(x_ref[...], w_ref[...], preferred_element_type=jnp.float32)
    o_ref[...] = h * dinv


_mm1 = pl.pallas_call(
    _mm1_body,
    grid=(GRID,),
    in_specs=[
        pl.BlockSpec((BR, D), lambda i: (i, 0)),
        pl.BlockSpec((D, D), lambda i: (0, 0)),
        pl.BlockSpec((BR, 2), lambda i: (i, 0)),
    ],
    out_specs=pl.BlockSpec((BR, D), lambda i: (i, 0)),
    out_shape=jax.ShapeDtypeStruct((N, D), jnp.float32),
)


def _mm2_body(a0_ref, a1_ref, h1_ref, deg_ref, b1_ref, w2_ref, o_ref):
    dinv = _dinv_of(deg_ref[...])
    t = (a0_ref[0] + a1_ref[0] + h1_ref[...]) * dinv + b1_ref[...]
    t = jnp.maximum(t, 0.0)
    o_ref[...] = jnp.dot(t, w2_ref[...], preferred_element_type=jnp.float32) * dinv


_mm2 = pl.pallas_call(
    _mm2_body,
    grid=(GRID,),
    in_specs=[
        pl.BlockSpec((1, BR, D), lambda i: (0, i, 0)),
        pl.BlockSpec((1, BR, D), lambda i: (1, i, 0)),
        pl.BlockSpec((BR, D), lambda i: (i, 0)),
        pl.BlockSpec((BR, 2), lambda i: (i, 0)),
        pl.BlockSpec((1, D), lambda i: (0, 0)),
        pl.BlockSpec((D, D), lambda i: (0, 0)),
    ],
    out_specs=pl.BlockSpec((BR, D), lambda i: (i, 0)),
    out_shape=jax.ShapeDtypeStruct((N, D), jnp.float32),
)


def _fin_body(a0_ref, a1_ref, h2_ref, deg_ref, b2_ref, o_ref):
    dinv = _dinv_of(deg_ref[...])
    o_ref[...] = (a0_ref[0] + a1_ref[0] + h2_ref[...]) * dinv + b2_ref[...]


_fin = pl.pallas_call(
    _fin_body,
    grid=(GRID,),
    in_specs=[
        pl.BlockSpec((1, BR, D), lambda i: (0, i, 0)),
        pl.BlockSpec((1, BR, D), lambda i: (1, i, 0)),
        pl.BlockSpec((BR, D), lambda i: (i, 0)),
        pl.BlockSpec((BR, 2), lambda i: (i, 0)),
        pl.BlockSpec((1, D), lambda i: (0, 0)),
    ],
    out_specs=pl.BlockSpec((BR, D), lambda i: (i, 0)),
    out_shape=jax.ShapeDtypeStruct((N, D), jnp.float32),
)


def kernel(node_emb, edge_index, W1, b1, W2, b2):
    src_p = edge_index[0].reshape(NW, EPT)
    dst_p = edge_index[1].reshape(NW, CHUNKS, K)

    deg_parts = _sc_deg(dst_p)
    deg_t = deg_parts.T

    b1r = b1.reshape(1, D)
    b2r = b2.reshape(1, D)

    h1p = _mm1(node_emb, W1, deg_t)
    agg1 = _sc_agg(h1p, src_p, dst_p)
    h2p = _mm2(agg1, agg1, h1p, deg_t, b1r, W2)
    agg2 = _sc_agg(h2p, src_p, dst_p)
    out = _fin(agg2, agg2, h2p, deg_t, b2r)
    return out


# speedup vs baseline: 1.9197x; 1.0227x over previous
import functools

import jax
import jax.numpy as jnp
from jax import lax
from jax.experimental import pallas as pl
from jax.experimental.pallas import tpu as pltpu
from jax.experimental.pallas import tpu_sc as plsc

N = 10000
E = 320000
D = 128

NC = 2
NS = 16
L = 16
NW = NC * NS

K = 80
CHUNKS = E // NW // K
EPT = K * CHUNKS

NPAD = 10112
ROWS_PER_TILE = NPAD // NS
NPAD1 = 10240
DEG_PER_TILE = NPAD1 // NS

_sc_mesh = plsc.VectorSubcoreMesh(
    core_axis_name="c", subcore_axis_name="s", num_cores=NC, num_subcores=NS
)


@functools.partial(
    pl.kernel,
    out_type=jax.ShapeDtypeStruct((NC, NPAD1), jnp.float32),
    mesh=_sc_mesh,
    scratch_types=[
        pltpu.VMEM((CHUNKS, K), jnp.int32),
        pltpu.VMEM((K,), jnp.float32),
        pltpu.VMEM((DEG_PER_TILE,), jnp.float32),
        pltpu.VMEM_SHARED((NPAD1,), jnp.float32),
    ],
)
def _sc_deg(dst_hbm, out_hbm, dst_v, ones_v, zeros_v, acc_sh):
    c = lax.axis_index("c")
    s = lax.axis_index("s")
    wid = s * NC + c
    pltpu.sync_copy(dst_hbm.at[wid], dst_v)

    def fill_ones(i, carry):
        ones_v[pl.ds(i * L, L)] = jnp.ones((L,), jnp.float32)
        return carry

    def fill_zeros(i, carry):
        zeros_v[pl.ds(i * L, L)] = jnp.zeros((L,), jnp.float32)
        return carry

    lax.fori_loop(0, K // L, fill_ones, 0)
    lax.fori_loop(0, DEG_PER_TILE // L, fill_zeros, 0)

    base = s * DEG_PER_TILE
    pltpu.sync_copy(zeros_v, acc_sh.at[pl.ds(base, DEG_PER_TILE)])
    plsc.subcore_barrier()

    def body(j, carry):
        pltpu.sync_copy(ones_v, acc_sh.at[dst_v.at[j]], add=True)
        return carry

    lax.fori_loop(0, CHUNKS, body, 0)
    plsc.subcore_barrier()
    pltpu.sync_copy(acc_sh.at[pl.ds(base, DEG_PER_TILE)],
                    out_hbm.at[c, pl.ds(base, DEG_PER_TILE)])


@functools.partial(
    pl.kernel,
    out_type=jax.ShapeDtypeStruct((NC, NPAD, D), jnp.float32),
    mesh=_sc_mesh,
    scratch_types=[
        pltpu.VMEM((EPT,), jnp.int32),
        pltpu.VMEM((CHUNKS, K), jnp.int32),
        pltpu.VMEM((K, D), jnp.float32),
        pltpu.VMEM((K, D), jnp.float32),
        pltpu.VMEM_SHARED((NPAD, D), jnp.float32),
        pltpu.SemaphoreType.DMA,
        pltpu.SemaphoreType.DMA,
        pltpu.SemaphoreType.DMA,
        pltpu.SemaphoreType.DMA,
    ],
)
def _sc_agg(h_hbm, src_hbm, dst_hbm, out_hbm, src_v, dst_v, rows0_v, rows1_v,
            acc_sh, sem0a, sem0b, sem1a, sem1b):
    c = lax.axis_index("c")
    s = lax.axis_index("s")
    wid = s * NC + c
    pltpu.sync_copy(src_hbm.at[wid], src_v)
    pltpu.sync_copy(dst_hbm.at[wid], dst_v)

    KH = K // 2

    def g_start(j, buf, sema, semb):
        pltpu.async_copy(h_hbm.at[src_v.at[pl.ds(j * K, KH)]],
                         buf.at[pl.ds(0, KH)], sema)
        pltpu.async_copy(h_hbm.at[src_v.at[pl.ds(j * K + KH, KH)]],
                         buf.at[pl.ds(KH, KH)], semb)

    def g_wait(j, buf, sema, semb):
        pltpu.make_async_copy(h_hbm.at[src_v.at[pl.ds(j * K, KH)]],
                              buf.at[pl.ds(0, KH)], sema).wait()
        pltpu.make_async_copy(h_hbm.at[src_v.at[pl.ds(j * K + KH, KH)]],
                              buf.at[pl.ds(KH, KH)], semb).wait()

    def scat(j, buf):
        pltpu.sync_copy(buf, acc_sh.at[dst_v.at[j]], add=True)

    g_start(0, rows0_v, sem0a, sem0b)

    def zrow(r, carry):
        for q in range(D // L):
            rows1_v[r, pl.ds(q * L, L)] = jnp.zeros((L,), jnp.float32)
        return carry

    lax.fori_loop(0, K, zrow, 0)

    base = s * ROWS_PER_TILE
    nfull = ROWS_PER_TILE // K
    for t in range(nfull):
        pltpu.sync_copy(rows1_v, acc_sh.at[pl.ds(base + t * K, K)])
    rem = ROWS_PER_TILE - nfull * K
    if rem:
        pltpu.sync_copy(rows1_v.at[pl.ds(0, rem)],
                        acc_sh.at[pl.ds(base + nfull * K, rem)])
    plsc.subcore_barrier()

    def body(p, carry):
        j0 = 2 * p
        j1 = j0 + 1
        g_start(j1, rows1_v, sem1a, sem1b)
        g_wait(j0, rows0_v, sem0a, sem0b)
        scat(j0, rows0_v)
        g_start(j0 + 2, rows0_v, sem0a, sem0b)
        g_wait(j1, rows1_v, sem1a, sem1b)
        scat(j1, rows1_v)
        return carry

    lax.fori_loop(0, CHUNKS // 2, body, 0)
    g_wait(CHUNKS - 1, rows0_v, sem0a, sem0b)
    scat(CHUNKS - 1, rows0_v)
    plsc.subcore_barrier()
    pltpu.sync_copy(acc_sh.at[pl.ds(base, ROWS_PER_TILE)],
                    out_hbm.at[c, pl.ds(base, ROWS_PER_TILE)])


BR = 1000
GRID = N // BR


def _dinv_of(deg_blk):
    return 1.0 / jnp.sqrt(deg_blk[:, 0:1] + deg_blk[:, 1:2] + 1.0)


def _mm1_body(x_ref, w_ref, deg_ref, o_ref):
    dinv = _dinv_of(deg_ref[...])
    h = jnp.dot(x_ref[...], w_ref[...], preferred_element_type=jnp.float32)
    o_ref[...] = h * dinv


_mm1 = pl.pallas_call(
    _mm1_body,
    grid=(GRID,),
    in_specs=[
        pl.BlockSpec((BR, D), lambda i: (i, 0)),
        pl.BlockSpec((D, D), lambda i: (0, 0)),
        pl.BlockSpec((BR, 2), lambda i: (i, 0)),
    ],
    out_specs=pl.BlockSpec((BR, D), lambda i: (i, 0)),
    out_shape=jax.ShapeDtypeStruct((N, D), jnp.float32),
)


def _mm2_body(a0_ref, a1_ref, h1_ref, deg_ref, b1_ref, w2_ref, o_ref):
    dinv = _dinv_of(deg_ref[...])
    t = (a0_ref[0] + a1_ref[0] + h1_ref[...]) * dinv + b1_ref[...]
    t = jnp.maximum(t, 0.0)
    o_ref[...] = jnp.dot(t, w2_ref[...], preferred_element_type=jnp.float32) * dinv


_mm2 = pl.pallas_call(
    _mm2_body,
    grid=(GRID,),
    in_specs=[
        pl.BlockSpec((1, BR, D), lambda i: (0, i, 0)),
        pl.BlockSpec((1, BR, D), lambda i: (1, i, 0)),
        pl.BlockSpec((BR, D), lambda i: (i, 0)),
        pl.BlockSpec((BR, 2), lambda i: (i, 0)),
        pl.BlockSpec((1, D), lambda i: (0, 0)),
        pl.BlockSpec((D, D), lambda i: (0, 0)),
    ],
    out_specs=pl.BlockSpec((BR, D), lambda i: (i, 0)),
    out_shape=jax.ShapeDtypeStruct((N, D), jnp.float32),
)


def _fin_body(a0_ref, a1_ref, h2_ref, deg_ref, b2_ref, o_ref):
    dinv = _dinv_of(deg_ref[...])
    o_ref[...] = (a0_ref[0] + a1_ref[0] + h2_ref[...]) * dinv + b2_ref[...]


_fin = pl.pallas_call(
    _fin_body,
    grid=(GRID,),
    in_specs=[
        pl.BlockSpec((1, BR, D), lambda i: (0, i, 0)),
        pl.BlockSpec((1, BR, D), lambda i: (1, i, 0)),
        pl.BlockSpec((BR, D), lambda i: (i, 0)),
        pl.BlockSpec((BR, 2), lambda i: (i, 0)),
        pl.BlockSpec((1, D), lambda i: (0, 0)),
    ],
    out_specs=pl.BlockSpec((BR, D), lambda i: (i, 0)),
    out_shape=jax.ShapeDtypeStruct((N, D), jnp.float32),
)


def kernel(node_emb, edge_index, W1, b1, W2, b2):
    src_p = edge_index[0].reshape(NW, EPT)
    dst_p = edge_index[1].reshape(NW, CHUNKS, K)

    deg_parts = _sc_deg(dst_p)
    deg_t = deg_parts.T

    b1r = b1.reshape(1, D)
    b2r = b2.reshape(1, D)

    h1p = _mm1(node_emb, W1, deg_t)
    agg1 = _sc_agg(h1p, src_p, dst_p)
    h2p = _mm2(agg1, agg1, h1p, deg_t, b1r, W2)
    agg2 = _sc_agg(h2p, src_p, dst_p)
    out = _fin(agg2, agg2, h2p, deg_t, b2r)
    return out


# speedup vs baseline: 1.9198x; 1.0001x over previous
import functools

import jax
import jax.numpy as jnp
from jax import lax
from jax.experimental import pallas as pl
from jax.experimental.pallas import tpu as pltpu
from jax.experimental.pallas import tpu_sc as plsc

N = 10000
E = 320000
D = 128

NC = 2
NS = 16
L = 16
NW = NC * NS

K = 80
CHUNKS = E // NW // K
EPT = K * CHUNKS

NPAD = 10112
ROWS_PER_TILE = NPAD // NS
NPAD1 = 10240
DEG_PER_TILE = NPAD1 // NS

_sc_mesh = plsc.VectorSubcoreMesh(
    core_axis_name="c", subcore_axis_name="s", num_cores=NC, num_subcores=NS
)


@functools.partial(
    pl.kernel,
    out_type=jax.ShapeDtypeStruct((NC, NPAD1), jnp.float32),
    mesh=_sc_mesh,
    scratch_types=[
        pltpu.VMEM((CHUNKS, K), jnp.int32),
        pltpu.VMEM((K,), jnp.float32),
        pltpu.VMEM((DEG_PER_TILE,), jnp.float32),
        pltpu.VMEM_SHARED((NPAD1,), jnp.float32),
    ],
)
def _sc_deg(dst_hbm, out_hbm, dst_v, ones_v, zeros_v, acc_sh):
    c = lax.axis_index("c")
    s = lax.axis_index("s")
    wid = s * NC + c
    pltpu.sync_copy(dst_hbm.at[wid], dst_v)

    def fill_ones(i, carry):
        ones_v[pl.ds(i * L, L)] = jnp.ones((L,), jnp.float32)
        return carry

    def fill_zeros(i, carry):
        zeros_v[pl.ds(i * L, L)] = jnp.zeros((L,), jnp.float32)
        return carry

    lax.fori_loop(0, K // L, fill_ones, 0)
    lax.fori_loop(0, DEG_PER_TILE // L, fill_zeros, 0)

    base = s * DEG_PER_TILE
    pltpu.sync_copy(zeros_v, acc_sh.at[pl.ds(base, DEG_PER_TILE)])
    plsc.subcore_barrier()

    def body(j, carry):
        pltpu.sync_copy(ones_v, acc_sh.at[dst_v.at[j]], add=True)
        return carry

    lax.fori_loop(0, CHUNKS, body, 0)
    plsc.subcore_barrier()
    pltpu.sync_copy(acc_sh.at[pl.ds(base, DEG_PER_TILE)],
                    out_hbm.at[c, pl.ds(base, DEG_PER_TILE)])


@functools.partial(
    pl.kernel,
    out_type=jax.ShapeDtypeStruct((NC, NPAD, D), jnp.float32),
    mesh=_sc_mesh,
    scratch_types=[
        pltpu.VMEM((EPT,), jnp.int32),
        pltpu.VMEM((CHUNKS, K), jnp.int32),
        pltpu.VMEM((K, D), jnp.float32),
        pltpu.VMEM((K, D), jnp.float32),
        pltpu.VMEM_SHARED((NPAD, D), jnp.float32),
        pltpu.SemaphoreType.DMA,
        pltpu.SemaphoreType.DMA,
        pltpu.SemaphoreType.DMA,
        pltpu.SemaphoreType.DMA,
    ],
)
def _sc_agg(h_hbm, src_hbm, dst_hbm, out_hbm, src_v, dst_v, rows0_v, rows1_v,
            acc_sh, sem0a, sem0b, sem1a, sem1b):
    c = lax.axis_index("c")
    s = lax.axis_index("s")
    wid = s * NC + c
    pltpu.sync_copy(src_hbm.at[wid], src_v)
    pltpu.sync_copy(dst_hbm.at[wid], dst_v)

    SPLITS = ((0, 24), (24, 24), (48, 16), (64, 16))

    def g_start(j, buf, sema, semb):
        for i, (off, ln) in enumerate(SPLITS):
            pltpu.async_copy(h_hbm.at[src_v.at[pl.ds(j * K + off, ln)]],
                             buf.at[pl.ds(off, ln)], sema if i % 2 == 0 else semb)

    def g_wait(j, buf, sema, semb):
        for i, (off, ln) in enumerate(SPLITS):
            pltpu.make_async_copy(h_hbm.at[src_v.at[pl.ds(j * K + off, ln)]],
                                  buf.at[pl.ds(off, ln)],
                                  sema if i % 2 == 0 else semb).wait()

    def scat(j, buf):
        pltpu.sync_copy(buf, acc_sh.at[dst_v.at[j]], add=True)

    g_start(0, rows0_v, sem0a, sem0b)

    def zrow(r, carry):
        for q in range(D // L):
            rows1_v[r, pl.ds(q * L, L)] = jnp.zeros((L,), jnp.float32)
        return carry

    lax.fori_loop(0, K, zrow, 0)

    base = s * ROWS_PER_TILE
    nfull = ROWS_PER_TILE // K
    for t in range(nfull):
        pltpu.sync_copy(rows1_v, acc_sh.at[pl.ds(base + t * K, K)])
    rem = ROWS_PER_TILE - nfull * K
    if rem:
        pltpu.sync_copy(rows1_v.at[pl.ds(0, rem)],
                        acc_sh.at[pl.ds(base + nfull * K, rem)])
    plsc.subcore_barrier()

    def body(p, carry):
        j0 = 2 * p
        j1 = j0 + 1
        g_start(j1, rows1_v, sem1a, sem1b)
        g_wait(j0, rows0_v, sem0a, sem0b)
        scat(j0, rows0_v)
        g_start(j0 + 2, rows0_v, sem0a, sem0b)
        g_wait(j1, rows1_v, sem1a, sem1b)
        scat(j1, rows1_v)
        return carry

    lax.fori_loop(0, CHUNKS // 2, body, 0)
    g_wait(CHUNKS - 1, rows0_v, sem0a, sem0b)
    scat(CHUNKS - 1, rows0_v)
    plsc.subcore_barrier()
    pltpu.sync_copy(acc_sh.at[pl.ds(base, ROWS_PER_TILE)],
                    out_hbm.at[c, pl.ds(base, ROWS_PER_TILE)])


BR = 1000
GRID = N // BR


def _dinv_of(deg_blk):
    return 1.0 / jnp.sqrt(deg_blk[:, 0:1] + deg_blk[:, 1:2] + 1.0)


def _mm1_body(x_ref, w_ref, deg_ref, o_ref):
    dinv = _dinv_of(deg_ref[...])
    h = jnp.dot(x_ref[...], w_ref[...], preferred_element_type=jnp.float32)
    o_ref[...] = h * dinv


_mm1 = pl.pallas_call(
    _mm1_body,
    grid=(GRID,),
    in_specs=[
        pl.BlockSpec((BR, D), lambda i: (i, 0)),
        pl.BlockSpec((D, D), lambda i: (0, 0)),
        pl.BlockSpec((BR, 2), lambda i: (i, 0)),
    ],
    out_specs=pl.BlockSpec((BR, D), lambda i: (i, 0)),
    out_shape=jax.ShapeDtypeStruct((N, D), jnp.float32),
)


def _mm2_body(a0_ref, a1_ref, h1_ref, deg_ref, b1_ref, w2_ref, o_ref):
    dinv = _dinv_of(deg_ref[...])
    t = (a0_ref[0] + a1_ref[0] + h1_ref[...]) * dinv + b1_ref[...]
    t = jnp.maximum(t, 0.0)
    o_ref[...] = jnp.dot(t, w2_ref[...], preferred_element_type=jnp.float32) * dinv


_mm2 = pl.pallas_call(
    _mm2_body,
    grid=(GRID,),
    in_specs=[
        pl.BlockSpec((1, BR, D), lambda i: (0, i, 0)),
        pl.BlockSpec((1, BR, D), lambda i: (1, i, 0)),
        pl.BlockSpec((BR, D), lambda i: (i, 0)),
        pl.BlockSpec((BR, 2), lambda i: (i, 0)),
        pl.BlockSpec((1, D), lambda i: (0, 0)),
        pl.BlockSpec((D, D), lambda i: (0, 0)),
    ],
    out_specs=pl.BlockSpec((BR, D), lambda i: (i, 0)),
    out_shape=jax.ShapeDtypeStruct((N, D), jnp.float32),
)


def _fin_body(a0_ref, a1_ref, h2_ref, deg_ref, b2_ref, o_ref):
    dinv = _dinv_of(deg_ref[...])
    o_ref[...] = (a0_ref[0] + a1_ref[0] + h2_ref[...]) * dinv + b2_ref[...]


_fin = pl.pallas_call(
    _fin_body,
    grid=(GRID,),
    in_specs=[
        pl.BlockSpec((1, BR, D), lambda i: (0, i, 0)),
        pl.BlockSpec((1, BR, D), lambda i: (1, i, 0)),
        pl.BlockSpec((BR, D), lambda i: (i, 0)),
        pl.BlockSpec((BR, 2), lambda i: (i, 0)),
        pl.BlockSpec((1, D), lambda i: (0, 0)),
    ],
    out_specs=pl.BlockSpec((BR, D), lambda i: (i, 0)),
    out_shape=jax.ShapeDtypeStruct((N, D), jnp.float32),
)


def kernel(node_emb, edge_index, W1, b1, W2, b2):
    src_p = edge_index[0].reshape(NW, EPT)
    dst_p = edge_index[1].reshape(NW, CHUNKS, K)

    deg_parts = _sc_deg(dst_p)
    deg_t = deg_parts.T

    b1r = b1.reshape(1, D)
    b2r = b2.reshape(1, D)

    h1p = _mm1(node_emb, W1, deg_t)
    agg1 = _sc_agg(h1p, src_p, dst_p)
    h2p = _mm2(agg1, agg1, h1p, deg_t, b1r, W2)
    agg2 = _sc_agg(h2p, src_p, dst_p)
    out = _fin(agg2, agg2, h2p, deg_t, b2r)
    return out
